# baseline (speedup 1.0000x reference)
import functools

import jax
import jax.numpy as jnp
from jax import lax
from jax.experimental import pallas as pl
from jax.experimental.pallas import tpu as pltpu
from jax.experimental.pallas import tpu_sc as plsc

N_NODES = 10000
D = 256
E = 160000

NW = 32
CH = 128
NCT = 1280
PADE = NCT * CH
K0 = 56
K1 = 24
KMAX = 56
PADC = 16 * K0 + 15 * K1 + KMAX
NR = 10240
STRIPE = NR // 16
HW = 64
NSL = 8
NBUF = 4
HB = NBUF // 2

_mesh = plsc.VectorSubcoreMesh(core_axis_name="c", subcore_axis_name="s")


@functools.partial(
    pl.kernel,
    mesh=_mesh,
    out_type=[jax.ShapeDtypeStruct((2, NR, HW), jnp.float32) for _ in range(NSL)]
    + [jax.ShapeDtypeStruct((2, NR, 16), jnp.float32)],
    scratch_types=[
        pltpu.VMEM((KMAX, CH), jnp.int32),
        pltpu.VMEM((KMAX, CH), jnp.int32),
        pltpu.VMEM((NBUF, CH, HW), jnp.float32),
        pltpu.VMEM((CH, HW), jnp.float32),
        pltpu.VMEM((CH, 16), jnp.float32),
        pltpu.VMEM((CH, 16), jnp.float32),
        pltpu.VMEM_SHARED((NR, HW), jnp.float32),
        pltpu.VMEM_SHARED((NR, 16), jnp.float32),
        pltpu.SemaphoreType.DMA((NBUF,)),
        pltpu.SemaphoreType.DMA((NBUF,)),
    ],
    compiler_params=pltpu.CompilerParams(use_tc_tiling_on_sc=False),
)
def _sc_segsum(src3, dst3, t0, t1, t2, t3, t4, t5, t6, t7, z128h, z16h, o16h,
               S0, S1, S2, S3, S4, S5, S6, S7, CNT,
               src_v, dst_v, rows_v, z128_v, z16_v, ones_v, acc, cnt_acc,
               g_sems, s_sems):
    c = lax.axis_index("c")
    s = lax.axis_index("s")
    r0 = s * STRIPE
    start = jnp.where(c == 0, s * K0, 16 * K0 + s * K1)
    nblk = jnp.where(c == 0, K0 // NBUF, K1 // NBUF)

    pltpu.sync_copy(src3.at[pl.ds(start, KMAX)], src_v)
    pltpu.sync_copy(dst3.at[pl.ds(start, KMAX)], dst_v)
    pltpu.sync_copy(z128h, z128_v)
    pltpu.sync_copy(z16h, z16_v)
    pltpu.sync_copy(o16h, ones_v)

    tabs = [t0, t1, t2, t3, t4, t5, t6, t7]
    outs = [S0, S1, S2, S3, S4, S5, S6, S7]
    for sl in range(NSL):
        for k in range(STRIPE // CH):
            pltpu.sync_copy(z128_v, acc.at[pl.ds(r0 + k * CH, CH)])
        if sl == 0:
            for k in range(STRIPE // CH):
                pltpu.sync_copy(z16_v, cnt_acc.at[pl.ds(r0 + k * CH, CH)])
        plsc.subcore_barrier()

        tab = tabs[sl]
        do_cnt = sl == 0

        def start_gather(ch, q):
            pltpu.async_copy(tab.at[src_v.at[ch]], rows_v.at[q], g_sems.at[q])

        def start_scatter(ch, q):
            pltpu.async_copy(rows_v.at[q], acc.at[dst_v.at[ch]], s_sems.at[q],
                             add=True)
            if do_cnt:
                pltpu.sync_copy(ones_v, cnt_acc.at[dst_v.at[ch]], add=True)

        def wait_gather(ch, q):
            pltpu.make_async_copy(tab.at[src_v.at[ch]], rows_v.at[q],
                                  g_sems.at[q]).wait()

        def wait_scatter(ch, q):
            pltpu.make_async_copy(rows_v.at[q], acc.at[dst_v.at[ch]],
                                  s_sems.at[q]).wait()

        def block(j, first):
            for p in range(2):
                for b in range(HB):
                    q = HB * p + b
                    ch = j * NBUF + q
                    if not first:
                        wait_scatter(ch - NBUF, q)
                    start_gather(ch, q)
                for b in range(HB):
                    q = HB * p + b
                    ch = j * NBUF + q
                    wait_gather(ch, q)
                    start_scatter(ch, q)

        block(0, True)
        lax.fori_loop(1, nblk, lambda j, cc: (block(j, False), cc)[1], 0)
        for q in range(NBUF):
            wait_scatter((nblk - 1) * NBUF + q, q)
        plsc.subcore_barrier()
        pltpu.sync_copy(acc.at[pl.ds(r0, STRIPE)],
                        outs[sl].at[c, pl.ds(r0, STRIPE)])
        if sl == 0:
            pltpu.sync_copy(cnt_acc.at[pl.ds(r0, STRIPE)],
                            CNT.at[c, pl.ds(r0, STRIPE)])


def _pre_body(u_ref, wupT_ref, bup_ref, ru_ref, uo_ref):
    u = u_ref[...]
    r = jnp.maximum(u, 0.0)
    ru_ref[...] = r
    uo_ref[...] = (
        jnp.dot(r, wupT_ref[...], preferred_element_type=jnp.float32)
        + bup_ref[...]
    )


def _post_body(s0_ref, s1_ref, s2_ref, s3_ref, s4_ref, s5_ref, s6_ref,
               s7_ref, cnt_ref, re_ref, rt_ref,
               wl1T_ref, wr1T_ref, b1_ref, wl2T_ref, wr2T_ref, b2_ref,
               wrpT_ref, brp_ref, out_ref):
    cnt = cnt_ref[0, :, 0:1] + cnt_ref[1, :, 0:1]
    inv = 1.0 / jnp.maximum(cnt, 1.0)
    m1 = jnp.concatenate(
        [s[0] + s[1] for s in (s0_ref, s1_ref, s2_ref, s3_ref)], axis=1) * inv
    m2 = jnp.concatenate(
        [s[0] + s[1] for s in (s4_ref, s5_ref, s6_ref, s7_ref)], axis=1) * inv
    r = re_ref[...] + rt_ref[...]
    f32 = jnp.float32
    r1 = jnp.maximum(
        jnp.dot(m1, wl1T_ref[...], preferred_element_type=f32) + b1_ref[...]
        + jnp.dot(r, wr1T_ref[...], preferred_element_type=f32), 0.0)
    r2 = (jnp.dot(m2, wl2T_ref[...], preferred_element_type=f32) + b2_ref[...]
          + jnp.dot(r1, wr2T_ref[...], preferred_element_type=f32))
    out_ref[...] = (
        jnp.dot(r2, wrpT_ref[...], preferred_element_type=f32) + brp_ref[...])


_B = 1000


def _full_spec():
    return pl.BlockSpec((D, D), lambda i: (0, 0))


def _bias_spec():
    return pl.BlockSpec((1, D), lambda i: (0, 0))


def kernel(edge_index_ur, edge_index_ri, ingredient_x, recipe_text_embeddings,
           user_emb, recipe_emb,
           W_l1_ur, W_r1_ur, b1_ur, W_l1_ri, W_r1_ri, b1_ri,
           W_l2_ur, W_r2_ur, b2_ur, W_l2_ri, W_r2_ri, b2_ri,
           W_up, b_up, W_rp, b_rp):
    src = edge_index_ur[0].astype(jnp.int32)
    dst = edge_index_ur[1].astype(jnp.int32)
    pad = PADC * CH - E
    src3 = jnp.concatenate([src, jnp.zeros((pad,), jnp.int32)]).reshape(
        PADC, CH)
    dst3 = jnp.concatenate([dst, jnp.full((pad,), N_NODES, jnp.int32)]
                           ).reshape(PADC, CH)

    ru, user_out = pl.pallas_call(
        _pre_body,
        grid=(N_NODES // _B,),
        in_specs=[
            pl.BlockSpec((_B, D), lambda i: (i, 0)),
            _full_spec(),
            _bias_spec(),
        ],
        out_specs=[
            pl.BlockSpec((_B, D), lambda i: (i, 0)),
            pl.BlockSpec((_B, D), lambda i: (i, 0)),
        ],
        out_shape=[
            jax.ShapeDtypeStruct((N_NODES, D), jnp.float32),
            jax.ShapeDtypeStruct((N_NODES, D), jnp.float32),
        ],
    )(user_emb, W_up.T, b_up.reshape(1, D))

    u_slices = [user_emb[:, j * HW:(j + 1) * HW] for j in range(4)]
    ru_slices = [ru[:, j * HW:(j + 1) * HW] for j in range(4)]
    z128 = jnp.zeros((CH, HW), jnp.float32)
    z16 = jnp.zeros((CH, 16), jnp.float32)
    o16 = jnp.ones((CH, 16), jnp.float32)

    *S, CNT = _sc_segsum(src3, dst3, *u_slices, *ru_slices, z128, z16, o16)

    part_spec = pl.BlockSpec((2, _B, HW), lambda i: (0, i, 0))
    recipe_out = pl.pallas_call(
        _post_body,
        grid=(N_NODES // _B,),
        in_specs=[
            part_spec, part_spec, part_spec, part_spec,
            part_spec, part_spec, part_spec, part_spec,
            pl.BlockSpec((2, _B, 16), lambda i: (0, i, 0)),
            pl.BlockSpec((_B, D), lambda i: (i, 0)),
            pl.BlockSpec((_B, D), lambda i: (i, 0)),
            _full_spec(), _full_spec(), _bias_spec(),
            _full_spec(), _full_spec(), _bias_spec(),
            _full_spec(), _bias_spec(),
        ],
        out_specs=pl.BlockSpec((_B, D), lambda i: (i, 0)),
        out_shape=jax.ShapeDtypeStruct((N_NODES, D), jnp.float32),
    )(*S, CNT, recipe_emb, recipe_text_embeddings,
      W_l1_ur.T, W_r1_ur.T, b1_ur.reshape(1, D),
      W_l2_ur.T, W_r2_ur.T, b2_ur.reshape(1, D),
      W_rp.T, b_rp.reshape(1, D))

    return user_out, recipe_out


# speedup vs baseline: 1.0159x; 1.0159x over previous
import functools

import jax
import jax.numpy as jnp
from jax import lax
from jax.experimental import pallas as pl
from jax.experimental.pallas import tpu as pltpu
from jax.experimental.pallas import tpu_sc as plsc

N_NODES = 10000
D = 256
E = 160000

NW = 32
CH = 128
NCT = 1280
PADE = NCT * CH
K0 = 60
K1 = 20
KMAX = 60
PADC = 16 * K0 + 15 * K1 + KMAX
NR = 10240
STRIPE = NR // 16
HW = 64
NSL = 8
NBUF = 4
HB = NBUF // 2

_mesh = plsc.VectorSubcoreMesh(core_axis_name="c", subcore_axis_name="s")


@functools.partial(
    pl.kernel,
    mesh=_mesh,
    out_type=[jax.ShapeDtypeStruct((2, NR, HW), jnp.float32) for _ in range(NSL)]
    + [jax.ShapeDtypeStruct((2, NR, 16), jnp.float32)],
    scratch_types=[
        pltpu.VMEM((KMAX, CH), jnp.int32),
        pltpu.VMEM((KMAX, CH), jnp.int32),
        pltpu.VMEM((NBUF, CH, HW), jnp.float32),
        pltpu.VMEM((CH, HW), jnp.float32),
        pltpu.VMEM((CH, 16), jnp.float32),
        pltpu.VMEM((CH, 16), jnp.float32),
        pltpu.VMEM_SHARED((NR, HW), jnp.float32),
        pltpu.VMEM_SHARED((NR, 16), jnp.float32),
        pltpu.SemaphoreType.DMA((NBUF,)),
        pltpu.SemaphoreType.DMA((NBUF,)),
    ],
    compiler_params=pltpu.CompilerParams(use_tc_tiling_on_sc=False),
)
def _sc_segsum(src3, dst3, t0, t1, t2, t3, t4, t5, t6, t7, z128h, z16h, o16h,
               S0, S1, S2, S3, S4, S5, S6, S7, CNT,
               src_v, dst_v, rows_v, z128_v, z16_v, ones_v, acc, cnt_acc,
               g_sems, s_sems):
    c = lax.axis_index("c")
    s = lax.axis_index("s")
    r0 = s * STRIPE
    start = jnp.where(c == 0, s * K0, 16 * K0 + s * K1)
    nblk = jnp.where(c == 0, K0 // NBUF, K1 // NBUF)

    pltpu.sync_copy(src3.at[pl.ds(start, KMAX)], src_v)
    pltpu.sync_copy(dst3.at[pl.ds(start, KMAX)], dst_v)
    pltpu.sync_copy(z128h, z128_v)
    pltpu.sync_copy(z16h, z16_v)
    pltpu.sync_copy(o16h, ones_v)

    tabs = [t0, t1, t2, t3, t4, t5, t6, t7]
    outs = [S0, S1, S2, S3, S4, S5, S6, S7]
    for sl in range(NSL):
        for k in range(STRIPE // CH):
            pltpu.sync_copy(z128_v, acc.at[pl.ds(r0 + k * CH, CH)])
        if sl == 0:
            for k in range(STRIPE // CH):
                pltpu.sync_copy(z16_v, cnt_acc.at[pl.ds(r0 + k * CH, CH)])
        plsc.subcore_barrier()

        tab = tabs[sl]
        do_cnt = sl == 0

        def start_gather(ch, q):
            pltpu.async_copy(tab.at[src_v.at[ch]], rows_v.at[q], g_sems.at[q])

        def start_scatter(ch, q):
            pltpu.async_copy(rows_v.at[q], acc.at[dst_v.at[ch]], s_sems.at[q],
                             add=True)
            if do_cnt:
                pltpu.sync_copy(ones_v, cnt_acc.at[dst_v.at[ch]], add=True)

        def wait_gather(ch, q):
            pltpu.make_async_copy(tab.at[src_v.at[ch]], rows_v.at[q],
                                  g_sems.at[q]).wait()

        def wait_scatter(ch, q):
            pltpu.make_async_copy(rows_v.at[q], acc.at[dst_v.at[ch]],
                                  s_sems.at[q]).wait()

        def block(j, first):
            for p in range(2):
                for b in range(HB):
                    q = HB * p + b
                    ch = j * NBUF + q
                    if not first:
                        wait_scatter(ch - NBUF, q)
                    start_gather(ch, q)
                for b in range(HB):
                    q = HB * p + b
                    ch = j * NBUF + q
                    wait_gather(ch, q)
                    start_scatter(ch, q)

        block(0, True)
        lax.fori_loop(1, nblk, lambda j, cc: (block(j, False), cc)[1], 0)
        for q in range(NBUF):
            wait_scatter((nblk - 1) * NBUF + q, q)
        plsc.subcore_barrier()
        pltpu.sync_copy(acc.at[pl.ds(r0, STRIPE)],
                        outs[sl].at[c, pl.ds(r0, STRIPE)])
        if sl == 0:
            pltpu.sync_copy(cnt_acc.at[pl.ds(r0, STRIPE)],
                            CNT.at[c, pl.ds(r0, STRIPE)])


def _pre_body(u_ref, wupT_ref, bup_ref, ru_ref, uo_ref):
    u = u_ref[...]
    r = jnp.maximum(u, 0.0)
    ru_ref[...] = r
    uo_ref[...] = (
        jnp.dot(r, wupT_ref[...], preferred_element_type=jnp.float32)
        + bup_ref[...]
    )


def _post_body(s0_ref, s1_ref, s2_ref, s3_ref, s4_ref, s5_ref, s6_ref,
               s7_ref, cnt_ref, re_ref, rt_ref,
               wl1T_ref, wr1T_ref, b1_ref, wl2T_ref, wr2T_ref, b2_ref,
               wrpT_ref, brp_ref, out_ref):
    cnt = cnt_ref[0, :, 0:1] + cnt_ref[1, :, 0:1]
    inv = 1.0 / jnp.maximum(cnt, 1.0)
    m1 = jnp.concatenate(
        [s[0] + s[1] for s in (s0_ref, s1_ref, s2_ref, s3_ref)], axis=1) * inv
    m2 = jnp.concatenate(
        [s[0] + s[1] for s in (s4_ref, s5_ref, s6_ref, s7_ref)], axis=1) * inv
    r = re_ref[...] + rt_ref[...]
    f32 = jnp.float32
    r1 = jnp.maximum(
        jnp.dot(m1, wl1T_ref[...], preferred_element_type=f32) + b1_ref[...]
        + jnp.dot(r, wr1T_ref[...], preferred_element_type=f32), 0.0)
    r2 = (jnp.dot(m2, wl2T_ref[...], preferred_element_type=f32) + b2_ref[...]
          + jnp.dot(r1, wr2T_ref[...], preferred_element_type=f32))
    out_ref[...] = (
        jnp.dot(r2, wrpT_ref[...], preferred_element_type=f32) + brp_ref[...])


_B = 1000


def _full_spec():
    return pl.BlockSpec((D, D), lambda i: (0, 0))


def _bias_spec():
    return pl.BlockSpec((1, D), lambda i: (0, 0))


def kernel(edge_index_ur, edge_index_ri, ingredient_x, recipe_text_embeddings,
           user_emb, recipe_emb,
           W_l1_ur, W_r1_ur, b1_ur, W_l1_ri, W_r1_ri, b1_ri,
           W_l2_ur, W_r2_ur, b2_ur, W_l2_ri, W_r2_ri, b2_ri,
           W_up, b_up, W_rp, b_rp):
    src = edge_index_ur[0].astype(jnp.int32)
    dst = edge_index_ur[1].astype(jnp.int32)
    pad = PADC * CH - E
    src3 = jnp.concatenate([src, jnp.zeros((pad,), jnp.int32)]).reshape(
        PADC, CH)
    dst3 = jnp.concatenate([dst, jnp.full((pad,), N_NODES, jnp.int32)]
                           ).reshape(PADC, CH)

    ru, user_out = pl.pallas_call(
        _pre_body,
        grid=(N_NODES // _B,),
        in_specs=[
            pl.BlockSpec((_B, D), lambda i: (i, 0)),
            _full_spec(),
            _bias_spec(),
        ],
        out_specs=[
            pl.BlockSpec((_B, D), lambda i: (i, 0)),
            pl.BlockSpec((_B, D), lambda i: (i, 0)),
        ],
        out_shape=[
            jax.ShapeDtypeStruct((N_NODES, D), jnp.float32),
            jax.ShapeDtypeStruct((N_NODES, D), jnp.float32),
        ],
    )(user_emb, W_up.T, b_up.reshape(1, D))

    u_slices = [user_emb[:, j * HW:(j + 1) * HW] for j in range(4)]
    ru_slices = [ru[:, j * HW:(j + 1) * HW] for j in range(4)]
    z128 = jnp.zeros((CH, HW), jnp.float32)
    z16 = jnp.zeros((CH, 16), jnp.float32)
    o16 = jnp.ones((CH, 16), jnp.float32)

    *S, CNT = _sc_segsum(src3, dst3, *u_slices, *ru_slices, z128, z16, o16)

    part_spec = pl.BlockSpec((2, _B, HW), lambda i: (0, i, 0))
    recipe_out = pl.pallas_call(
        _post_body,
        grid=(N_NODES // _B,),
        in_specs=[
            part_spec, part_spec, part_spec, part_spec,
            part_spec, part_spec, part_spec, part_spec,
            pl.BlockSpec((2, _B, 16), lambda i: (0, i, 0)),
            pl.BlockSpec((_B, D), lambda i: (i, 0)),
            pl.BlockSpec((_B, D), lambda i: (i, 0)),
            _full_spec(), _full_spec(), _bias_spec(),
            _full_spec(), _full_spec(), _bias_spec(),
            _full_spec(), _bias_spec(),
        ],
        out_specs=pl.BlockSpec((_B, D), lambda i: (i, 0)),
        out_shape=jax.ShapeDtypeStruct((N_NODES, D), jnp.float32),
    )(*S, CNT, recipe_emb, recipe_text_embeddings,
      W_l1_ur.T, W_r1_ur.T, b1_ur.reshape(1, D),
      W_l2_ur.T, W_r2_ur.T, b2_ur.reshape(1, D),
      W_rp.T, b_rp.reshape(1, D))

    return user_out, recipe_out


# speedup vs baseline: 1.1043x; 1.0870x over previous
import functools

import jax
import jax.numpy as jnp
from jax import lax
from jax.experimental import pallas as pl
from jax.experimental.pallas import tpu as pltpu
from jax.experimental.pallas import tpu_sc as plsc

N_NODES = 10000
D = 256
E = 160000

NW = 32
CH = 128
NCT = 1280
PADE = NCT * CH
K0 = 68
K1 = 12
KMAX = 68
PADC = 16 * K0 + 15 * K1 + KMAX
NR = 10240
STRIPE = NR // 16
HW = 64
NSL = 8
NBUF = 4
HB = NBUF // 2

_mesh = plsc.VectorSubcoreMesh(core_axis_name="c", subcore_axis_name="s")


@functools.partial(
    pl.kernel,
    mesh=_mesh,
    out_type=[jax.ShapeDtypeStruct((2, NR, HW), jnp.float32) for _ in range(NSL)]
    + [jax.ShapeDtypeStruct((2, NR, 16), jnp.float32)],
    scratch_types=[
        pltpu.VMEM((KMAX, CH), jnp.int32),
        pltpu.VMEM((KMAX, CH), jnp.int32),
        pltpu.VMEM((NBUF, CH, HW), jnp.float32),
        pltpu.VMEM((CH, HW), jnp.float32),
        pltpu.VMEM((CH, 16), jnp.float32),
        pltpu.VMEM((CH, 16), jnp.float32),
        pltpu.VMEM_SHARED((NR, HW), jnp.float32),
        pltpu.VMEM_SHARED((NR, 16), jnp.float32),
        pltpu.SemaphoreType.DMA((NBUF,)),
        pltpu.SemaphoreType.DMA((NBUF,)),
    ],
    compiler_params=pltpu.CompilerParams(use_tc_tiling_on_sc=False),
)
def _sc_segsum(src3, dst3, t0, t1, t2, t3, t4, t5, t6, t7, z128h, z16h, o16h,
               S0, S1, S2, S3, S4, S5, S6, S7, CNT,
               src_v, dst_v, rows_v, z128_v, z16_v, ones_v, acc, cnt_acc,
               g_sems, s_sems):
    c = lax.axis_index("c")
    s = lax.axis_index("s")
    r0 = s * STRIPE
    start = jnp.where(c == 0, s * K0, 16 * K0 + s * K1)
    nblk = jnp.where(c == 0, K0 // NBUF, K1 // NBUF)

    pltpu.sync_copy(src3.at[pl.ds(start, KMAX)], src_v)
    pltpu.sync_copy(dst3.at[pl.ds(start, KMAX)], dst_v)
    pltpu.sync_copy(z128h, z128_v)
    pltpu.sync_copy(z16h, z16_v)
    pltpu.sync_copy(o16h, ones_v)

    tabs = [t0, t1, t2, t3, t4, t5, t6, t7]
    outs = [S0, S1, S2, S3, S4, S5, S6, S7]
    for sl in range(NSL):
        for k in range(STRIPE // CH):
            pltpu.sync_copy(z128_v, acc.at[pl.ds(r0 + k * CH, CH)])
        if sl == 0:
            for k in range(STRIPE // CH):
                pltpu.sync_copy(z16_v, cnt_acc.at[pl.ds(r0 + k * CH, CH)])
        plsc.subcore_barrier()

        tab = tabs[sl]
        do_cnt = sl == 0

        def start_gather(ch, q):
            pltpu.async_copy(tab.at[src_v.at[ch]], rows_v.at[q], g_sems.at[q])

        def start_scatter(ch, q):
            pltpu.async_copy(rows_v.at[q], acc.at[dst_v.at[ch]], s_sems.at[q],
                             add=True)
            if do_cnt:
                pltpu.sync_copy(ones_v, cnt_acc.at[dst_v.at[ch]], add=True)

        def wait_gather(ch, q):
            pltpu.make_async_copy(tab.at[src_v.at[ch]], rows_v.at[q],
                                  g_sems.at[q]).wait()

        def wait_scatter(ch, q):
            pltpu.make_async_copy(rows_v.at[q], acc.at[dst_v.at[ch]],
                                  s_sems.at[q]).wait()

        def block(j, first):
            for p in range(2):
                for b in range(HB):
                    q = HB * p + b
                    ch = j * NBUF + q
                    if not first:
                        wait_scatter(ch - NBUF, q)
                    start_gather(ch, q)
                for b in range(HB):
                    q = HB * p + b
                    ch = j * NBUF + q
                    wait_gather(ch, q)
                    start_scatter(ch, q)

        block(0, True)
        lax.fori_loop(1, nblk, lambda j, cc: (block(j, False), cc)[1], 0)
        for q in range(NBUF):
            wait_scatter((nblk - 1) * NBUF + q, q)
        plsc.subcore_barrier()
        pltpu.sync_copy(acc.at[pl.ds(r0, STRIPE)],
                        outs[sl].at[c, pl.ds(r0, STRIPE)])
        if sl == 0:
            pltpu.sync_copy(cnt_acc.at[pl.ds(r0, STRIPE)],
                            CNT.at[c, pl.ds(r0, STRIPE)])


def _pre_body(u_ref, wupT_ref, bup_ref, ru_ref, uo_ref):
    u = u_ref[...]
    r = jnp.maximum(u, 0.0)
    ru_ref[...] = r
    uo_ref[...] = (
        jnp.dot(r, wupT_ref[...], preferred_element_type=jnp.float32)
        + bup_ref[...]
    )


def _post_body(s0_ref, s1_ref, s2_ref, s3_ref, s4_ref, s5_ref, s6_ref,
               s7_ref, cnt_ref, re_ref, rt_ref,
               wl1T_ref, wr1T_ref, b1_ref, wl2T_ref, wr2T_ref, b2_ref,
               wrpT_ref, brp_ref, out_ref):
    cnt = cnt_ref[0, :, 0:1] + cnt_ref[1, :, 0:1]
    inv = 1.0 / jnp.maximum(cnt, 1.0)
    m1 = jnp.concatenate(
        [s[0] + s[1] for s in (s0_ref, s1_ref, s2_ref, s3_ref)], axis=1) * inv
    m2 = jnp.concatenate(
        [s[0] + s[1] for s in (s4_ref, s5_ref, s6_ref, s7_ref)], axis=1) * inv
    r = re_ref[...] + rt_ref[...]
    f32 = jnp.float32
    r1 = jnp.maximum(
        jnp.dot(m1, wl1T_ref[...], preferred_element_type=f32) + b1_ref[...]
        + jnp.dot(r, wr1T_ref[...], preferred_element_type=f32), 0.0)
    r2 = (jnp.dot(m2, wl2T_ref[...], preferred_element_type=f32) + b2_ref[...]
          + jnp.dot(r1, wr2T_ref[...], preferred_element_type=f32))
    out_ref[...] = (
        jnp.dot(r2, wrpT_ref[...], preferred_element_type=f32) + brp_ref[...])


_B = 1000


def _full_spec():
    return pl.BlockSpec((D, D), lambda i: (0, 0))


def _bias_spec():
    return pl.BlockSpec((1, D), lambda i: (0, 0))


def kernel(edge_index_ur, edge_index_ri, ingredient_x, recipe_text_embeddings,
           user_emb, recipe_emb,
           W_l1_ur, W_r1_ur, b1_ur, W_l1_ri, W_r1_ri, b1_ri,
           W_l2_ur, W_r2_ur, b2_ur, W_l2_ri, W_r2_ri, b2_ri,
           W_up, b_up, W_rp, b_rp):
    src = edge_index_ur[0].astype(jnp.int32)
    dst = edge_index_ur[1].astype(jnp.int32)
    pad = PADC * CH - E
    src3 = jnp.concatenate([src, jnp.zeros((pad,), jnp.int32)]).reshape(
        PADC, CH)
    dst3 = jnp.concatenate([dst, jnp.full((pad,), N_NODES, jnp.int32)]
                           ).reshape(PADC, CH)

    ru, user_out = pl.pallas_call(
        _pre_body,
        grid=(N_NODES // _B,),
        in_specs=[
            pl.BlockSpec((_B, D), lambda i: (i, 0)),
            _full_spec(),
            _bias_spec(),
        ],
        out_specs=[
            pl.BlockSpec((_B, D), lambda i: (i, 0)),
            pl.BlockSpec((_B, D), lambda i: (i, 0)),
        ],
        out_shape=[
            jax.ShapeDtypeStruct((N_NODES, D), jnp.float32),
            jax.ShapeDtypeStruct((N_NODES, D), jnp.float32),
        ],
    )(user_emb, W_up.T, b_up.reshape(1, D))

    u_slices = [user_emb[:, j * HW:(j + 1) * HW] for j in range(4)]
    ru_slices = [ru[:, j * HW:(j + 1) * HW] for j in range(4)]
    z128 = jnp.zeros((CH, HW), jnp.float32)
    z16 = jnp.zeros((CH, 16), jnp.float32)
    o16 = jnp.ones((CH, 16), jnp.float32)

    *S, CNT = _sc_segsum(src3, dst3, *u_slices, *ru_slices, z128, z16, o16)

    part_spec = pl.BlockSpec((2, _B, HW), lambda i: (0, i, 0))
    recipe_out = pl.pallas_call(
        _post_body,
        grid=(N_NODES // _B,),
        in_specs=[
            part_spec, part_spec, part_spec, part_spec,
            part_spec, part_spec, part_spec, part_spec,
            pl.BlockSpec((2, _B, 16), lambda i: (0, i, 0)),
            pl.BlockSpec((_B, D), lambda i: (i, 0)),
            pl.BlockSpec((_B, D), lambda i: (i, 0)),
            _full_spec(), _full_spec(), _bias_spec(),
            _full_spec(), _full_spec(), _bias_spec(),
            _full_spec(), _bias_spec(),
        ],
        out_specs=pl.BlockSpec((_B, D), lambda i: (i, 0)),
        out_shape=jax.ShapeDtypeStruct((N_NODES, D), jnp.float32),
    )(*S, CNT, recipe_emb, recipe_text_embeddings,
      W_l1_ur.T, W_r1_ur.T, b1_ur.reshape(1, D),
      W_l2_ur.T, W_r2_ur.T, b2_ur.reshape(1, D),
      W_rp.T, b_rp.reshape(1, D))

    return user_out, recipe_out


# speedup vs baseline: 1.1702x; 1.0596x over previous
import functools

import jax
import jax.numpy as jnp
from jax import lax
from jax.experimental import pallas as pl
from jax.experimental.pallas import tpu as pltpu
from jax.experimental.pallas import tpu_sc as plsc

N_NODES = 10000
D = 256
E = 160000

NW = 32
CH = 128
NCT = 1280
PADE = NCT * CH
K0 = 72
K1 = 8
KMAX = 72
PADC = 16 * K0 + 15 * K1 + KMAX
NR = 10240
STRIPE = NR // 16
HW = 64
NSL = 8
NBUF = 4
HB = NBUF // 2

_mesh = plsc.VectorSubcoreMesh(core_axis_name="c", subcore_axis_name="s")


@functools.partial(
    pl.kernel,
    mesh=_mesh,
    out_type=[jax.ShapeDtypeStruct((2, NR, HW), jnp.float32) for _ in range(NSL)]
    + [jax.ShapeDtypeStruct((2, NR, 16), jnp.float32)],
    scratch_types=[
        pltpu.VMEM((KMAX, CH), jnp.int32),
        pltpu.VMEM((KMAX, CH), jnp.int32),
        pltpu.VMEM((NBUF, CH, HW), jnp.float32),
        pltpu.VMEM((CH, HW), jnp.float32),
        pltpu.VMEM((CH, 16), jnp.float32),
        pltpu.VMEM((CH, 16), jnp.float32),
        pltpu.VMEM_SHARED((NR, HW), jnp.float32),
        pltpu.VMEM_SHARED((NR, 16), jnp.float32),
        pltpu.SemaphoreType.DMA((NBUF,)),
        pltpu.SemaphoreType.DMA((NBUF,)),
    ],
    compiler_params=pltpu.CompilerParams(use_tc_tiling_on_sc=False),
)
def _sc_segsum(src3, dst3, t0, t1, t2, t3, t4, t5, t6, t7, z128h, z16h, o16h,
               S0, S1, S2, S3, S4, S5, S6, S7, CNT,
               src_v, dst_v, rows_v, z128_v, z16_v, ones_v, acc, cnt_acc,
               g_sems, s_sems):
    c = lax.axis_index("c")
    s = lax.axis_index("s")
    r0 = s * STRIPE
    start = jnp.where(c == 0, s * K0, 16 * K0 + s * K1)
    nblk = jnp.where(c == 0, K0 // NBUF, K1 // NBUF)

    pltpu.sync_copy(src3.at[pl.ds(start, KMAX)], src_v)
    pltpu.sync_copy(dst3.at[pl.ds(start, KMAX)], dst_v)
    pltpu.sync_copy(z128h, z128_v)
    pltpu.sync_copy(z16h, z16_v)
    pltpu.sync_copy(o16h, ones_v)

    tabs = [t0, t1, t2, t3, t4, t5, t6, t7]
    outs = [S0, S1, S2, S3, S4, S5, S6, S7]
    for sl in range(NSL):
        for k in range(STRIPE // CH):
            pltpu.sync_copy(z128_v, acc.at[pl.ds(r0 + k * CH, CH)])
        if sl == 0:
            for k in range(STRIPE // CH):
                pltpu.sync_copy(z16_v, cnt_acc.at[pl.ds(r0 + k * CH, CH)])
        plsc.subcore_barrier()

        tab = tabs[sl]
        do_cnt = sl == 0

        def start_gather(ch, q):
            pltpu.async_copy(tab.at[src_v.at[ch]], rows_v.at[q], g_sems.at[q])

        def start_scatter(ch, q):
            pltpu.async_copy(rows_v.at[q], acc.at[dst_v.at[ch]], s_sems.at[q],
                             add=True)
            if do_cnt:
                pltpu.sync_copy(ones_v, cnt_acc.at[dst_v.at[ch]], add=True)

        def wait_gather(ch, q):
            pltpu.make_async_copy(tab.at[src_v.at[ch]], rows_v.at[q],
                                  g_sems.at[q]).wait()

        def wait_scatter(ch, q):
            pltpu.make_async_copy(rows_v.at[q], acc.at[dst_v.at[ch]],
                                  s_sems.at[q]).wait()

        def block(j, first):
            for p in range(2):
                for b in range(HB):
                    q = HB * p + b
                    ch = j * NBUF + q
                    if not first:
                        wait_scatter(ch - NBUF, q)
                    start_gather(ch, q)
                for b in range(HB):
                    q = HB * p + b
                    ch = j * NBUF + q
                    wait_gather(ch, q)
                    start_scatter(ch, q)

        block(0, True)
        lax.fori_loop(1, nblk, lambda j, cc: (block(j, False), cc)[1], 0)
        for q in range(NBUF):
            wait_scatter((nblk - 1) * NBUF + q, q)
        plsc.subcore_barrier()
        pltpu.sync_copy(acc.at[pl.ds(r0, STRIPE)],
                        outs[sl].at[c, pl.ds(r0, STRIPE)])
        if sl == 0:
            pltpu.sync_copy(cnt_acc.at[pl.ds(r0, STRIPE)],
                            CNT.at[c, pl.ds(r0, STRIPE)])


def _pre_body(u_ref, wupT_ref, bup_ref, ru_ref, uo_ref):
    u = u_ref[...]
    r = jnp.maximum(u, 0.0)
    ru_ref[...] = r
    uo_ref[...] = (
        jnp.dot(r, wupT_ref[...], preferred_element_type=jnp.float32)
        + bup_ref[...]
    )


def _post_body(s0_ref, s1_ref, s2_ref, s3_ref, s4_ref, s5_ref, s6_ref,
               s7_ref, cnt_ref, re_ref, rt_ref,
               wl1T_ref, wr1T_ref, b1_ref, wl2T_ref, wr2T_ref, b2_ref,
               wrpT_ref, brp_ref, out_ref):
    cnt = cnt_ref[0, :, 0:1] + cnt_ref[1, :, 0:1]
    inv = 1.0 / jnp.maximum(cnt, 1.0)
    m1 = jnp.concatenate(
        [s[0] + s[1] for s in (s0_ref, s1_ref, s2_ref, s3_ref)], axis=1) * inv
    m2 = jnp.concatenate(
        [s[0] + s[1] for s in (s4_ref, s5_ref, s6_ref, s7_ref)], axis=1) * inv
    r = re_ref[...] + rt_ref[...]
    f32 = jnp.float32
    r1 = jnp.maximum(
        jnp.dot(m1, wl1T_ref[...], preferred_element_type=f32) + b1_ref[...]
        + jnp.dot(r, wr1T_ref[...], preferred_element_type=f32), 0.0)
    r2 = (jnp.dot(m2, wl2T_ref[...], preferred_element_type=f32) + b2_ref[...]
          + jnp.dot(r1, wr2T_ref[...], preferred_element_type=f32))
    out_ref[...] = (
        jnp.dot(r2, wrpT_ref[...], preferred_element_type=f32) + brp_ref[...])


_B = 1000


def _full_spec():
    return pl.BlockSpec((D, D), lambda i: (0, 0))


def _bias_spec():
    return pl.BlockSpec((1, D), lambda i: (0, 0))


def kernel(edge_index_ur, edge_index_ri, ingredient_x, recipe_text_embeddings,
           user_emb, recipe_emb,
           W_l1_ur, W_r1_ur, b1_ur, W_l1_ri, W_r1_ri, b1_ri,
           W_l2_ur, W_r2_ur, b2_ur, W_l2_ri, W_r2_ri, b2_ri,
           W_up, b_up, W_rp, b_rp):
    src = edge_index_ur[0].astype(jnp.int32)
    dst = edge_index_ur[1].astype(jnp.int32)
    pad = PADC * CH - E
    src3 = jnp.concatenate([src, jnp.zeros((pad,), jnp.int32)]).reshape(
        PADC, CH)
    dst3 = jnp.concatenate([dst, jnp.full((pad,), N_NODES, jnp.int32)]
                           ).reshape(PADC, CH)

    ru, user_out = pl.pallas_call(
        _pre_body,
        grid=(N_NODES // _B,),
        in_specs=[
            pl.BlockSpec((_B, D), lambda i: (i, 0)),
            _full_spec(),
            _bias_spec(),
        ],
        out_specs=[
            pl.BlockSpec((_B, D), lambda i: (i, 0)),
            pl.BlockSpec((_B, D), lambda i: (i, 0)),
        ],
        out_shape=[
            jax.ShapeDtypeStruct((N_NODES, D), jnp.float32),
            jax.ShapeDtypeStruct((N_NODES, D), jnp.float32),
        ],
    )(user_emb, W_up.T, b_up.reshape(1, D))

    u_slices = [user_emb[:, j * HW:(j + 1) * HW] for j in range(4)]
    ru_slices = [ru[:, j * HW:(j + 1) * HW] for j in range(4)]
    z128 = jnp.zeros((CH, HW), jnp.float32)
    z16 = jnp.zeros((CH, 16), jnp.float32)
    o16 = jnp.ones((CH, 16), jnp.float32)

    *S, CNT = _sc_segsum(src3, dst3, *u_slices, *ru_slices, z128, z16, o16)

    part_spec = pl.BlockSpec((2, _B, HW), lambda i: (0, i, 0))
    recipe_out = pl.pallas_call(
        _post_body,
        grid=(N_NODES // _B,),
        in_specs=[
            part_spec, part_spec, part_spec, part_spec,
            part_spec, part_spec, part_spec, part_spec,
            pl.BlockSpec((2, _B, 16), lambda i: (0, i, 0)),
            pl.BlockSpec((_B, D), lambda i: (i, 0)),
            pl.BlockSpec((_B, D), lambda i: (i, 0)),
            _full_spec(), _full_spec(), _bias_spec(),
            _full_spec(), _full_spec(), _bias_spec(),
            _full_spec(), _bias_spec(),
        ],
        out_specs=pl.BlockSpec((_B, D), lambda i: (i, 0)),
        out_shape=jax.ShapeDtypeStruct((N_NODES, D), jnp.float32),
    )(*S, CNT, recipe_emb, recipe_text_embeddings,
      W_l1_ur.T, W_r1_ur.T, b1_ur.reshape(1, D),
      W_l2_ur.T, W_r2_ur.T, b2_ur.reshape(1, D),
      W_rp.T, b_rp.reshape(1, D))

    return user_out, recipe_out


# speedup vs baseline: 1.1744x; 1.0036x over previous
import functools

import jax
import jax.numpy as jnp
from jax import lax
from jax.experimental import pallas as pl
from jax.experimental.pallas import tpu as pltpu
from jax.experimental.pallas import tpu_sc as plsc

N_NODES = 10000
D = 256
E = 160000

NW = 32
CH = 128
NCT = 1280
PADE = NCT * CH
K0 = 76
K1 = 4
KMAX = 76
PADC = 16 * K0 + 15 * K1 + KMAX
NR = 10240
STRIPE = NR // 16
HW = 64
NSL = 8
NBUF = 4
HB = NBUF // 2

_mesh = plsc.VectorSubcoreMesh(core_axis_name="c", subcore_axis_name="s")


@functools.partial(
    pl.kernel,
    mesh=_mesh,
    out_type=[jax.ShapeDtypeStruct((2, NR, HW), jnp.float32) for _ in range(NSL)]
    + [jax.ShapeDtypeStruct((2, NR, 16), jnp.float32)],
    scratch_types=[
        pltpu.VMEM((KMAX, CH), jnp.int32),
        pltpu.VMEM((KMAX, CH), jnp.int32),
        pltpu.VMEM((NBUF, CH, HW), jnp.float32),
        pltpu.VMEM((CH, HW), jnp.float32),
        pltpu.VMEM((CH, 16), jnp.float32),
        pltpu.VMEM((CH, 16), jnp.float32),
        pltpu.VMEM_SHARED((NR, HW), jnp.float32),
        pltpu.VMEM_SHARED((NR, 16), jnp.float32),
        pltpu.SemaphoreType.DMA((NBUF,)),
        pltpu.SemaphoreType.DMA((NBUF,)),
    ],
    compiler_params=pltpu.CompilerParams(use_tc_tiling_on_sc=False),
)
def _sc_segsum(src3, dst3, t0, t1, t2, t3, t4, t5, t6, t7, z128h, z16h, o16h,
               S0, S1, S2, S3, S4, S5, S6, S7, CNT,
               src_v, dst_v, rows_v, z128_v, z16_v, ones_v, acc, cnt_acc,
               g_sems, s_sems):
    c = lax.axis_index("c")
    s = lax.axis_index("s")
    r0 = s * STRIPE
    start = jnp.where(c == 0, s * K0, 16 * K0 + s * K1)
    nblk = jnp.where(c == 0, K0 // NBUF, K1 // NBUF)

    pltpu.sync_copy(src3.at[pl.ds(start, KMAX)], src_v)
    pltpu.sync_copy(dst3.at[pl.ds(start, KMAX)], dst_v)
    pltpu.sync_copy(z128h, z128_v)
    pltpu.sync_copy(z16h, z16_v)
    pltpu.sync_copy(o16h, ones_v)

    tabs = [t0, t1, t2, t3, t4, t5, t6, t7]
    outs = [S0, S1, S2, S3, S4, S5, S6, S7]
    for sl in range(NSL):
        for k in range(STRIPE // CH):
            pltpu.sync_copy(z128_v, acc.at[pl.ds(r0 + k * CH, CH)])
        if sl == 0:
            for k in range(STRIPE // CH):
                pltpu.sync_copy(z16_v, cnt_acc.at[pl.ds(r0 + k * CH, CH)])
        plsc.subcore_barrier()

        tab = tabs[sl]
        do_cnt = sl == 0

        def start_gather(ch, q):
            pltpu.async_copy(tab.at[src_v.at[ch]], rows_v.at[q], g_sems.at[q])

        def start_scatter(ch, q):
            pltpu.async_copy(rows_v.at[q], acc.at[dst_v.at[ch]], s_sems.at[q],
                             add=True)
            if do_cnt:
                pltpu.sync_copy(ones_v, cnt_acc.at[dst_v.at[ch]], add=True)

        def wait_gather(ch, q):
            pltpu.make_async_copy(tab.at[src_v.at[ch]], rows_v.at[q],
                                  g_sems.at[q]).wait()

        def wait_scatter(ch, q):
            pltpu.make_async_copy(rows_v.at[q], acc.at[dst_v.at[ch]],
                                  s_sems.at[q]).wait()

        def block(j, first):
            for p in range(2):
                for b in range(HB):
                    q = HB * p + b
                    ch = j * NBUF + q
                    if not first:
                        wait_scatter(ch - NBUF, q)
                    start_gather(ch, q)
                for b in range(HB):
                    q = HB * p + b
                    ch = j * NBUF + q
                    wait_gather(ch, q)
                    start_scatter(ch, q)

        block(0, True)
        lax.fori_loop(1, nblk, lambda j, cc: (block(j, False), cc)[1], 0)
        for q in range(NBUF):
            wait_scatter((nblk - 1) * NBUF + q, q)
        plsc.subcore_barrier()
        pltpu.sync_copy(acc.at[pl.ds(r0, STRIPE)],
                        outs[sl].at[c, pl.ds(r0, STRIPE)])
        if sl == 0:
            pltpu.sync_copy(cnt_acc.at[pl.ds(r0, STRIPE)],
                            CNT.at[c, pl.ds(r0, STRIPE)])


def _pre_body(u_ref, wupT_ref, bup_ref, ru_ref, uo_ref):
    u = u_ref[...]
    r = jnp.maximum(u, 0.0)
    ru_ref[...] = r
    uo_ref[...] = (
        jnp.dot(r, wupT_ref[...], preferred_element_type=jnp.float32)
        + bup_ref[...]
    )


def _post_body(s0_ref, s1_ref, s2_ref, s3_ref, s4_ref, s5_ref, s6_ref,
               s7_ref, cnt_ref, re_ref, rt_ref,
               wl1T_ref, wr1T_ref, b1_ref, wl2T_ref, wr2T_ref, b2_ref,
               wrpT_ref, brp_ref, out_ref):
    cnt = cnt_ref[0, :, 0:1] + cnt_ref[1, :, 0:1]
    inv = 1.0 / jnp.maximum(cnt, 1.0)
    m1 = jnp.concatenate(
        [s[0] + s[1] for s in (s0_ref, s1_ref, s2_ref, s3_ref)], axis=1) * inv
    m2 = jnp.concatenate(
        [s[0] + s[1] for s in (s4_ref, s5_ref, s6_ref, s7_ref)], axis=1) * inv
    r = re_ref[...] + rt_ref[...]
    f32 = jnp.float32
    r1 = jnp.maximum(
        jnp.dot(m1, wl1T_ref[...], preferred_element_type=f32) + b1_ref[...]
        + jnp.dot(r, wr1T_ref[...], preferred_element_type=f32), 0.0)
    r2 = (jnp.dot(m2, wl2T_ref[...], preferred_element_type=f32) + b2_ref[...]
          + jnp.dot(r1, wr2T_ref[...], preferred_element_type=f32))
    out_ref[...] = (
        jnp.dot(r2, wrpT_ref[...], preferred_element_type=f32) + brp_ref[...])


_B = 1000


def _full_spec():
    return pl.BlockSpec((D, D), lambda i: (0, 0))


def _bias_spec():
    return pl.BlockSpec((1, D), lambda i: (0, 0))


def kernel(edge_index_ur, edge_index_ri, ingredient_x, recipe_text_embeddings,
           user_emb, recipe_emb,
           W_l1_ur, W_r1_ur, b1_ur, W_l1_ri, W_r1_ri, b1_ri,
           W_l2_ur, W_r2_ur, b2_ur, W_l2_ri, W_r2_ri, b2_ri,
           W_up, b_up, W_rp, b_rp):
    src = edge_index_ur[0].astype(jnp.int32)
    dst = edge_index_ur[1].astype(jnp.int32)
    pad = PADC * CH - E
    src3 = jnp.concatenate([src, jnp.zeros((pad,), jnp.int32)]).reshape(
        PADC, CH)
    dst3 = jnp.concatenate([dst, jnp.full((pad,), N_NODES, jnp.int32)]
                           ).reshape(PADC, CH)

    ru, user_out = pl.pallas_call(
        _pre_body,
        grid=(N_NODES // _B,),
        in_specs=[
            pl.BlockSpec((_B, D), lambda i: (i, 0)),
            _full_spec(),
            _bias_spec(),
        ],
        out_specs=[
            pl.BlockSpec((_B, D), lambda i: (i, 0)),
            pl.BlockSpec((_B, D), lambda i: (i, 0)),
        ],
        out_shape=[
            jax.ShapeDtypeStruct((N_NODES, D), jnp.float32),
            jax.ShapeDtypeStruct((N_NODES, D), jnp.float32),
        ],
    )(user_emb, W_up.T, b_up.reshape(1, D))

    u_slices = [user_emb[:, j * HW:(j + 1) * HW] for j in range(4)]
    ru_slices = [ru[:, j * HW:(j + 1) * HW] for j in range(4)]
    z128 = jnp.zeros((CH, HW), jnp.float32)
    z16 = jnp.zeros((CH, 16), jnp.float32)
    o16 = jnp.ones((CH, 16), jnp.float32)

    *S, CNT = _sc_segsum(src3, dst3, *u_slices, *ru_slices, z128, z16, o16)

    part_spec = pl.BlockSpec((2, _B, HW), lambda i: (0, i, 0))
    recipe_out = pl.pallas_call(
        _post_body,
        grid=(N_NODES // _B,),
        in_specs=[
            part_spec, part_spec, part_spec, part_spec,
            part_spec, part_spec, part_spec, part_spec,
            pl.BlockSpec((2, _B, 16), lambda i: (0, i, 0)),
            pl.BlockSpec((_B, D), lambda i: (i, 0)),
            pl.BlockSpec((_B, D), lambda i: (i, 0)),
            _full_spec(), _full_spec(), _bias_spec(),
            _full_spec(), _full_spec(), _bias_spec(),
            _full_spec(), _bias_spec(),
        ],
        out_specs=pl.BlockSpec((_B, D), lambda i: (i, 0)),
        out_shape=jax.ShapeDtypeStruct((N_NODES, D), jnp.float32),
    )(*S, CNT, recipe_emb, recipe_text_embeddings,
      W_l1_ur.T, W_r1_ur.T, b1_ur.reshape(1, D),
      W_l2_ur.T, W_r2_ur.T, b2_ur.reshape(1, D),
      W_rp.T, b_rp.reshape(1, D))

    return user_out, recipe_out
